# baseline (speedup 1.0000x reference)
import functools

import jax
import jax.numpy as jnp
from jax import lax
from jax.experimental import pallas as pl
from jax.experimental.pallas import tpu as pltpu
from jax.experimental.pallas import tpu_sc as plsc

BSZ = 32
BEAM = 4
VOCAB = 100000
PAD = 1
EOS = 2
UNK = 3
WORD_REWARD = 0.5
UNK_REWARD = -1.0
CAND_SIZE = 2 * BEAM
NEG_INF = -1e9

ROWS = BSZ * BEAM
NW = 32
RPW = ROWS // NW
CHUNK = 20000
NCHUNK = VOCAB // CHUNK
SEGV = 125
SEGW = SEGV * 16
SPC = CHUNK // SEGW
NSEG = VOCAB // SEGW
UNR = 25
CCAP = 1024
I32MAX = 2**31 - 1
F32NINF = float("-inf")


def _sc_topk_body(lg_hbm, ov_hbm, oi_hbm, buf, segmax, cav, cai, outv, outi,
                  sem):
    cid = lax.axis_index("c")
    sid = lax.axis_index("s")
    wid = sid * 2 + cid
    lanes = lax.broadcasted_iota(jnp.int32, (16,), 0)
    adj = jnp.where(lanes == PAD, NEG_INF,
                    jnp.where(lanes == EOS, -WORD_REWARD,
                              jnp.where(lanes == UNK, UNK_REWARD, 0.0)))
    ninf16 = jnp.full((16,), F32NINF, jnp.float32)
    zero16 = jnp.zeros((16,), jnp.int32)
    row0 = wid * RPW

    def src(r, c):
        start = pl.multiple_of((row0 + r) * VOCAB + c * CHUNK, 8)
        return lg_hbm.at[pl.ds(start, CHUNK)]

    def issue(r, c):
        return pltpu.async_copy(src(r, c), buf.at[pl.ds(c * CHUNK, CHUNK)],
                                sem)

    def issue_row(r):
        def ic(c, _):
            issue(r, c)
            return 0
        lax.fori_loop(0, NCHUNK, ic, 0)

    issue_row(0)

    def row_body(r, _):
        def clr(j, c):
            cav[pl.ds(j * 16, 16)] = ninf16
            return c
        lax.fori_loop(0, (CCAP + 16) // 16, clr, 0)

        def chunk_body(c, gmax):
            pltpu.make_async_copy(src(r, c), buf.at[pl.ds(c * CHUNK, CHUNK)],
                                  sem).wait()

            @pl.when(c == 0)
            def _():
                buf[pl.ds(0, 16)] = buf[pl.ds(0, 16)] + adj

            def seg_body(s, gm):
                sb = c * CHUNK + s * SEGW

                def inner(j, m):
                    b = sb + j * (UNR * 16)
                    for u in range(UNR):
                        m = jnp.maximum(m, buf[pl.ds(b + u * 16, 16)])
                    return m
                m = lax.fori_loop(0, SEGV // UNR, inner, ninf16)
                segmax[pl.ds((c * SPC + s) * 16, 16)] = m
                return jnp.maximum(gm, m)
            return lax.fori_loop(0, SPC, seg_body, gmax)
        gmax = lax.fori_loop(0, NCHUNK, chunk_body, ninf16)

        sk, _ = plsc.sort_key_val(gmax, gmax, descending=True)
        thr = jnp.max(jnp.where(lanes == CAND_SIZE - 1, sk, F32NINF))

        def seg_scan(s, off):
            smax = jnp.max(segmax[pl.ds(s * 16, 16)])

            def rescan(off):
                def inner(j, o):
                    b = s * SEGW + j * (UNR * 16)
                    for u in range(UNR):
                        v = buf[pl.ds(b + u * 16, 16)]
                        pc = plsc.all_reduce_population_count(v >= thr)
                        hit = pc > 0
                        okm = hit & (o < CCAP)
                        pos = o + lanes
                        iv = lanes + (b + u * 16)
                        plsc.store_scatter(cav, [pos], v, mask=okm)
                        plsc.store_scatter(cai, [pos], iv, mask=okm)
                        o = o + jnp.where(okm, 16, 0)
                    return o
                return lax.fori_loop(0, SEGV // UNR, inner, off)
            return lax.cond(smax >= thr, rescan, lambda o: o, off)
        off = lax.fori_loop(0, NSEG, seg_scan, zero16)

        @pl.when(r < RPW - 1)
        def _():
            issue_row(r + 1)

        cnt = jnp.minimum(jnp.max(off), CCAP)
        nv = (cnt + 15) // 16
        vals8 = ninf16
        idx8 = zero16
        for k in range(CAND_SIZE):
            def ea(j, g):
                return jnp.maximum(g, cav[pl.ds(j * 16, 16)])
            gm = jnp.max(lax.fori_loop(0, nv, ea, ninf16))

            def eb(j, w):
                v = cav[pl.ds(j * 16, 16)]
                iv = cai[pl.ds(j * 16, 16)]
                return jnp.minimum(w, jnp.where(v == gm, iv, I32MAX))
            wi = jnp.min(lax.fori_loop(
                0, nv, eb, jnp.full((16,), I32MAX, jnp.int32)))

            def ec(j, c):
                v = cav[pl.ds(j * 16, 16)]
                iv = cai[pl.ds(j * 16, 16)]
                cav[pl.ds(j * 16, 16)] = jnp.where(iv == wi, F32NINF, v)
                return c
            lax.fori_loop(0, nv, ec, 0)
            vals8 = jnp.where(lanes == k, gm, vals8)
            idx8 = jnp.where(lanes == k, wi, idx8)
        m8 = lanes < CAND_SIZE
        plsc.store_compressed(outv.at[pl.ds(r * CAND_SIZE, 16)], vals8,
                              mask=m8)
        plsc.store_compressed(outi.at[pl.ds(r * CAND_SIZE, 16)], idx8,
                              mask=m8)
        return 0
    lax.fori_loop(0, RPW, row_body, 0)

    nout = RPW * CAND_SIZE
    pltpu.sync_copy(outv.at[pl.ds(0, nout)],
                    ov_hbm.at[pl.ds(wid * nout, nout)])
    pltpu.sync_copy(outi.at[pl.ds(0, nout)],
                    oi_hbm.at[pl.ds(wid * nout, nout)])


def _sc_topk(lg2):
    call = pl.kernel(
        _sc_topk_body,
        out_type=[
            jax.ShapeDtypeStruct((ROWS * CAND_SIZE,), jnp.float32),
            jax.ShapeDtypeStruct((ROWS * CAND_SIZE,), jnp.int32),
        ],
        name="sc_topk",
        mesh=plsc.VectorSubcoreMesh(core_axis_name="c", subcore_axis_name="s",
                                    num_cores=2, num_subcores=16),
        compiler_params=pltpu.CompilerParams(needs_layout_passes=False,
                                             use_tc_tiling_on_sc=False),
        scratch_types=[
            pltpu.VMEM((VOCAB,), jnp.float32),
            pltpu.VMEM((NSEG * 16,), jnp.float32),
            pltpu.VMEM((CCAP + 16,), jnp.float32),
            pltpu.VMEM((CCAP + 16,), jnp.int32),
            pltpu.VMEM((RPW * CAND_SIZE + 16,), jnp.float32),
            pltpu.VMEM((RPW * CAND_SIZE + 16,), jnp.int32),
            pltpu.SemaphoreType.DMA,
        ],
    )
    return call(lg2)


_LSE_RB = 8


def _lse_body(x_ref, o_ref):
    x = x_ref[0]
    m = jnp.max(x, axis=1, keepdims=True)
    se = jnp.sum(jnp.exp(x - m), axis=1, keepdims=True)
    o_ref[...] = (m + jnp.log(se)).reshape(1, _LSE_RB, 1)


def _lse(lg3):
    nb = ROWS // _LSE_RB
    return pl.pallas_call(
        _lse_body,
        grid=(nb,),
        in_specs=[pl.BlockSpec((1, _LSE_RB, VOCAB), lambda i: (i, 0, 0))],
        out_specs=pl.BlockSpec((1, _LSE_RB, 1), lambda i: (i, 0, 0)),
        out_shape=jax.ShapeDtypeStruct((nb, _LSE_RB, 1), jnp.float32),
    )(lg3)


def _merge_body(v_ref, i_ref, s_ref, l_ref, vals_ref, idx_ref, beams_ref,
                toks_ref, eos_ref):
    y = v_ref[...]
    ii = i_ref[...]
    s = s_ref[...]
    l = l_ref[...]
    col = lax.broadcasted_iota(jnp.int32, (BSZ, BEAM * CAND_SIZE), 1)
    beam = col // CAND_SIZE
    b = y + WORD_REWARD + s - l
    flat = beam * VOCAB + ii
    lane8 = lax.broadcasted_iota(jnp.int32, (BSZ, CAND_SIZE), 1)
    ovals = jnp.zeros((BSZ, CAND_SIZE), jnp.float32)
    oidx = jnp.zeros((BSZ, CAND_SIZE), jnp.int32)
    for k in range(CAND_SIZE):
        gm = jnp.max(b, axis=1, keepdims=True)
        win = jnp.min(jnp.where(b == gm, flat, I32MAX), axis=1, keepdims=True)
        b = jnp.where(flat == win, F32NINF, b)
        ovals = jnp.where(lane8 == k, gm, ovals)
        oidx = jnp.where(lane8 == k, win, oidx)
    toks = oidx % VOCAB
    vals_ref[...] = ovals
    idx_ref[...] = oidx
    beams_ref[...] = oidx // VOCAB
    toks_ref[...] = toks
    eos_ref[...] = jnp.where(toks == EOS, jnp.int32(1), jnp.int32(0))


def _merge(yv, yi, s_rep, l_rep):
    shp8 = jax.ShapeDtypeStruct((BSZ, CAND_SIZE), jnp.float32)
    shp8i = jax.ShapeDtypeStruct((BSZ, CAND_SIZE), jnp.int32)
    return pl.pallas_call(
        _merge_body,
        out_shape=(shp8, shp8i, shp8i, shp8i, shp8i),
    )(yv, yi, s_rep, l_rep)


def kernel(logits, scores):
    lg3 = logits.reshape(ROWS // _LSE_RB, _LSE_RB, VOCAB)

    yv, yi = _sc_topk(logits.reshape(ROWS * VOCAB))
    lse = _lse(lg3)

    yv = yv.reshape(BSZ, BEAM * CAND_SIZE)
    yi = yi.reshape(BSZ, BEAM * CAND_SIZE)
    s_rep = jnp.broadcast_to(scores.reshape(BSZ, BEAM, 1),
                             (BSZ, BEAM, CAND_SIZE)).reshape(
                                 BSZ, BEAM * CAND_SIZE)
    l_rep = jnp.broadcast_to(lse.reshape(BSZ, BEAM, 1),
                             (BSZ, BEAM, CAND_SIZE)).reshape(
                                 BSZ, BEAM * CAND_SIZE)

    vals, idx, beams, toks, eos = _merge(yv, yi, s_rep, l_rep)
    return vals, idx, beams, toks, eos.astype(bool)


# speedup vs baseline: 1.0029x; 1.0029x over previous
import functools

import jax
import jax.numpy as jnp
from jax import lax
from jax.experimental import pallas as pl
from jax.experimental.pallas import tpu as pltpu
from jax.experimental.pallas import tpu_sc as plsc

BSZ = 32
BEAM = 4
VOCAB = 100000
PAD = 1
EOS = 2
UNK = 3
WORD_REWARD = 0.5
UNK_REWARD = -1.0
CAND_SIZE = 2 * BEAM
NEG_INF = -1e9

ROWS = BSZ * BEAM
NW = 32
RPW = ROWS // NW
CHUNK = 20000
NCHUNK = VOCAB // CHUNK
SEGV = 125
SEGW = SEGV * 16
SPC = CHUNK // SEGW
NSEG = VOCAB // SEGW
UNR = 25
CCAP = 1024
I32MAX = 2**31 - 1
F32NINF = float("-inf")


def _sc_topk_body(lg_hbm, ov_hbm, oi_hbm, buf, segmax, cav, cai, outv, outi,
                  sem):
    cid = lax.axis_index("c")
    sid = lax.axis_index("s")
    wid = sid * 2 + cid
    lanes = lax.broadcasted_iota(jnp.int32, (16,), 0)
    adj = jnp.where(lanes == PAD, NEG_INF,
                    jnp.where(lanes == EOS, -WORD_REWARD,
                              jnp.where(lanes == UNK, UNK_REWARD, 0.0)))
    ninf16 = jnp.full((16,), F32NINF, jnp.float32)
    zero16 = jnp.zeros((16,), jnp.int32)
    row0 = wid * RPW

    def src(r, c):
        start = pl.multiple_of((row0 + r) * VOCAB + c * CHUNK, 8)
        return lg_hbm.at[pl.ds(start, CHUNK)]

    def issue(r, c):
        return pltpu.async_copy(src(r, c), buf.at[pl.ds(c * CHUNK, CHUNK)],
                                sem)

    def issue_row(r):
        def ic(c, _):
            issue(r, c)
            return 0
        lax.fori_loop(0, NCHUNK, ic, 0)

    issue_row(0)

    def row_body(r, _):
        def clr(j, c):
            cav[pl.ds(j * 16, 16)] = ninf16
            return c
        lax.fori_loop(0, (CCAP + 16) // 16, clr, 0)

        def chunk_body(c, gmax):
            pltpu.make_async_copy(src(r, c), buf.at[pl.ds(c * CHUNK, CHUNK)],
                                  sem).wait()

            @pl.when(c == 0)
            def _():
                buf[pl.ds(0, 16)] = buf[pl.ds(0, 16)] + adj

            def seg_body(s, gm):
                sb = c * CHUNK + s * SEGW

                def inner(j, m):
                    b = sb + j * (UNR * 16)
                    for u in range(UNR):
                        m = jnp.maximum(m, buf[pl.ds(b + u * 16, 16)])
                    return m
                m = lax.fori_loop(0, SEGV // UNR, inner, ninf16)
                segmax[pl.ds((c * SPC + s) * 16, 16)] = m
                return jnp.maximum(gm, m)
            return lax.fori_loop(0, SPC, seg_body, gmax)
        gmax = lax.fori_loop(0, NCHUNK, chunk_body, ninf16)

        sk, _ = plsc.sort_key_val(gmax, gmax, descending=True)
        thr = jnp.max(jnp.where(lanes == CAND_SIZE - 1, sk, F32NINF))

        def seg_scan(s, off):
            smax = jnp.max(segmax[pl.ds(s * 16, 16)])

            def rescan(off):
                def inner(j, o):
                    b = s * SEGW + j * (UNR * 16)
                    for u in range(UNR):
                        v = buf[pl.ds(b + u * 16, 16)]
                        pc = plsc.all_reduce_population_count(v >= thr)
                        hit = pc > 0
                        okm = hit & (o < CCAP)
                        pos = o + lanes
                        iv = lanes + (b + u * 16)
                        plsc.store_scatter(cav, [pos], v, mask=okm)
                        plsc.store_scatter(cai, [pos], iv, mask=okm)
                        o = o + jnp.where(okm, 16, 0)
                    return o
                return lax.fori_loop(0, SEGV // UNR, inner, off)
            return lax.cond(smax >= thr, rescan, lambda o: o, off)
        off = lax.fori_loop(0, NSEG, seg_scan, zero16)

        @pl.when(r < RPW - 1)
        def _():
            issue_row(r + 1)

        cnt = jnp.minimum(jnp.max(off), CCAP)
        nv = (cnt + 15) // 16
        vals8 = ninf16
        idx8 = zero16
        for k in range(CAND_SIZE):
            def ea(j, g):
                return jnp.maximum(g, cav[pl.ds(j * 16, 16)])
            gm = jnp.max(lax.fori_loop(0, nv, ea, ninf16))

            def eb(j, w):
                v = cav[pl.ds(j * 16, 16)]
                iv = cai[pl.ds(j * 16, 16)]
                return jnp.minimum(w, jnp.where(v == gm, iv, I32MAX))
            wi = jnp.min(lax.fori_loop(
                0, nv, eb, jnp.full((16,), I32MAX, jnp.int32)))

            def ec(j, c):
                v = cav[pl.ds(j * 16, 16)]
                iv = cai[pl.ds(j * 16, 16)]
                cav[pl.ds(j * 16, 16)] = jnp.where(iv == wi, F32NINF, v)
                return c
            lax.fori_loop(0, nv, ec, 0)
            vals8 = jnp.where(lanes == k, gm, vals8)
            idx8 = jnp.where(lanes == k, wi, idx8)
        m8 = lanes < CAND_SIZE
        plsc.store_compressed(outv.at[pl.ds(r * CAND_SIZE, 16)], vals8,
                              mask=m8)
        plsc.store_compressed(outi.at[pl.ds(r * CAND_SIZE, 16)], idx8,
                              mask=m8)
        return 0
    lax.fori_loop(0, RPW, row_body, 0)

    nout = RPW * CAND_SIZE
    pltpu.sync_copy(outv.at[pl.ds(0, nout)],
                    ov_hbm.at[pl.ds(wid * nout, nout)])
    pltpu.sync_copy(outi.at[pl.ds(0, nout)],
                    oi_hbm.at[pl.ds(wid * nout, nout)])


def _sc_topk(lg2):
    call = pl.kernel(
        _sc_topk_body,
        out_type=[
            jax.ShapeDtypeStruct((ROWS * CAND_SIZE,), jnp.float32),
            jax.ShapeDtypeStruct((ROWS * CAND_SIZE,), jnp.int32),
        ],
        name="sc_topk",
        mesh=plsc.VectorSubcoreMesh(core_axis_name="c", subcore_axis_name="s",
                                    num_cores=2, num_subcores=16),
        compiler_params=pltpu.CompilerParams(needs_layout_passes=False),
        scratch_types=[
            pltpu.VMEM((VOCAB,), jnp.float32),
            pltpu.VMEM((NSEG * 16,), jnp.float32),
            pltpu.VMEM((CCAP + 16,), jnp.float32),
            pltpu.VMEM((CCAP + 16,), jnp.int32),
            pltpu.VMEM((RPW * CAND_SIZE + 16,), jnp.float32),
            pltpu.VMEM((RPW * CAND_SIZE + 16,), jnp.int32),
            pltpu.SemaphoreType.DMA,
        ],
    )
    return call(lg2)


_LSE_RB = 8


def _lse_body(x_ref, o_ref):
    x = x_ref[0]
    m = jnp.max(x, axis=1, keepdims=True)
    se = jnp.sum(jnp.exp(x - m), axis=1, keepdims=True)
    o_ref[...] = (m + jnp.log(se)).reshape(1, _LSE_RB, 1)


def _lse(lg3):
    nb = ROWS // _LSE_RB
    return pl.pallas_call(
        _lse_body,
        grid=(nb,),
        in_specs=[pl.BlockSpec((1, _LSE_RB, VOCAB), lambda i: (i, 0, 0))],
        out_specs=pl.BlockSpec((1, _LSE_RB, 1), lambda i: (i, 0, 0)),
        out_shape=jax.ShapeDtypeStruct((nb, _LSE_RB, 1), jnp.float32),
    )(lg3)


def _merge_body(v_ref, i_ref, s_ref, l_ref, vals_ref, idx_ref, beams_ref,
                toks_ref, eos_ref):
    y = v_ref[...]
    ii = i_ref[...]
    s = s_ref[...]
    l = l_ref[...]
    col = lax.broadcasted_iota(jnp.int32, (BSZ, BEAM * CAND_SIZE), 1)
    beam = col // CAND_SIZE
    b = y + WORD_REWARD + s - l
    flat = beam * VOCAB + ii
    lane8 = lax.broadcasted_iota(jnp.int32, (BSZ, CAND_SIZE), 1)
    ovals = jnp.zeros((BSZ, CAND_SIZE), jnp.float32)
    oidx = jnp.zeros((BSZ, CAND_SIZE), jnp.int32)
    for k in range(CAND_SIZE):
        gm = jnp.max(b, axis=1, keepdims=True)
        win = jnp.min(jnp.where(b == gm, flat, I32MAX), axis=1, keepdims=True)
        b = jnp.where(flat == win, F32NINF, b)
        ovals = jnp.where(lane8 == k, gm, ovals)
        oidx = jnp.where(lane8 == k, win, oidx)
    toks = oidx % VOCAB
    vals_ref[...] = ovals
    idx_ref[...] = oidx
    beams_ref[...] = oidx // VOCAB
    toks_ref[...] = toks
    eos_ref[...] = jnp.where(toks == EOS, jnp.int32(1), jnp.int32(0))


def _merge(yv, yi, s_rep, l_rep):
    shp8 = jax.ShapeDtypeStruct((BSZ, CAND_SIZE), jnp.float32)
    shp8i = jax.ShapeDtypeStruct((BSZ, CAND_SIZE), jnp.int32)
    return pl.pallas_call(
        _merge_body,
        out_shape=(shp8, shp8i, shp8i, shp8i, shp8i),
    )(yv, yi, s_rep, l_rep)


def kernel(logits, scores):
    lg3 = logits.reshape(ROWS // _LSE_RB, _LSE_RB, VOCAB)

    yv, yi = _sc_topk(logits.reshape(ROWS * VOCAB))
    lse = _lse(lg3)

    yv = yv.reshape(BSZ, BEAM * CAND_SIZE)
    yi = yi.reshape(BSZ, BEAM * CAND_SIZE)
    s_rep = jnp.broadcast_to(scores.reshape(BSZ, BEAM, 1),
                             (BSZ, BEAM, CAND_SIZE)).reshape(
                                 BSZ, BEAM * CAND_SIZE)
    l_rep = jnp.broadcast_to(lse.reshape(BSZ, BEAM, 1),
                             (BSZ, BEAM, CAND_SIZE)).reshape(
                                 BSZ, BEAM * CAND_SIZE)

    vals, idx, beams, toks, eos = _merge(yv, yi, s_rep, l_rep)
    return vals, idx, beams, toks, eos.astype(bool)
